# baseline (speedup 1.0000x reference)
import jax
import jax.numpy as jnp
from jax import lax
from jax.experimental import pallas as pl
from jax.experimental.pallas import tpu as pltpu
from jax.experimental.pallas import tpu_sc as plsc

_B, _ROWS, _COLS = 4, 512, 512
_NV = 5
_EXT0, _EXT1 = -51.2, -51.2
_GRID_R = 102.4 / _ROWS
_GRID_C = 102.4 / _COLS
_THRESH = 0.05

_NC, _NS, _L = 2, 16, 16
_NW = _NC * _NS
_ROWS_PER_W = (_B * _ROWS) // _NW
_RC = 8
_NCHUNK = _ROWS_PER_W // _RC
_WPB = _ROWS // _ROWS_PER_W


def _rsqrt(x):
    i = lax.bitcast_convert_type(x, jnp.int32)
    i = jnp.int32(0x5F3759DF) - lax.shift_right_logical(i, 1)
    y = lax.bitcast_convert_type(i, jnp.float32)
    for _ in range(2):
        y = y * (1.5 - 0.5 * x * y * y)
    return y


def _decode_body(voxel, pixel, conf, off, view0, view1, view2, view3, view4,
                 vel, out,
                 voxel_b, pixel_b, conf_b, off0_b, off1_b, vel0_b, vel1_b,
                 view_b, out_b, ccol_b):
    views = (view0, view1, view2, view3, view4)
    wid = lax.axis_index("s") * _NC + lax.axis_index("c")
    b = wid // _WPB
    r_base = (wid % _WPB) * _ROWS_PER_W

    iota = lax.iota(jnp.int32, _L)
    iota_f = iota.astype(jnp.float32)

    def ccol_body(g, c):
        cb = g * _L
        ccol_b[pl.ds(cb, _L)] = _EXT1 + (cb + iota_f + 0.5) * _GRID_C
        return c
    lax.fori_loop(0, _COLS // _L, ccol_body, 0)

    def chunk_body(ci, carry):
        r0 = r_base + ci * _RC
        pltpu.sync_copy(voxel.at[b, pl.ds(r0, _RC)], voxel_b)
        pltpu.sync_copy(pixel.at[b, pl.ds(r0, _RC)], pixel_b)
        pltpu.sync_copy(conf.at[b, pl.ds(r0, _RC)], conf_b)
        pltpu.sync_copy(off.at[b, 0, pl.ds(r0, _RC)], off0_b)
        pltpu.sync_copy(off.at[b, 1, pl.ds(r0, _RC)], off1_b)
        pltpu.sync_copy(vel.at[b, 0, pl.ds(r0, _RC)], vel0_b)
        pltpu.sync_copy(vel.at[b, 1, pl.ds(r0, _RC)], vel1_b)
        for v in range(_NV):
            pltpu.sync_copy(views[v].at[b, pl.ds(r0, _RC)], view_b.at[v])

        for row in range(_RC):
            r_glob = (r0 + row).astype(jnp.float32)
            cr_base = _EXT0 + (r_glob + 0.5) * _GRID_R
            row_splat = jnp.full((_L,), row, jnp.int32)

            def group_body(g, carry2):
                cbase = g * _L
                c_vec = cbase + iota
                s = jnp.zeros((_L,), jnp.float32)
                for v in range(_NV):
                    idx_v = view_b[v, row, pl.ds(cbase, _L)]
                    s = s + plsc.load_gather(conf_b, [row_splat, idx_v])
                conf_lin = conf_b[row, pl.ds(cbase, _L)]
                conf_final = 0.5 * (conf_lin + s * (1.0 / _NV))

                center_r = cr_base + off0_b[row, pl.ds(cbase, _L)] * _GRID_R
                center_c = (ccol_b[pl.ds(cbase, _L)]
                            + off1_b[row, pl.ds(cbase, _L)] * _GRID_C)

                vx = vel0_b[row, pl.ds(cbase, _L)]
                vy = vel1_b[row, pl.ds(cbase, _L)]
                s2 = vx * vx + vy * vy + 1e-12
                speed = s2 * _rsqrt(s2)

                mask = ((pixel_b[row, pl.ds(cbase, _L)] > _THRESH)
                        & (voxel_b[row, pl.ds(cbase, _L)] > 0))
                neg = jnp.full((_L,), -0.1, jnp.float32)
                for ch, val in enumerate((center_r, center_c, conf_final,
                                          speed)):
                    out_b[ch, row, pl.ds(cbase, _L)] = jnp.where(mask, val,
                                                                 neg)
                return carry2

            lax.fori_loop(0, _COLS // _L, group_body, 0, unroll=4)

        for ch in range(4):
            pltpu.sync_copy(out_b.at[ch], out.at[ch, b, pl.ds(r0, _RC)])
        return carry

    lax.fori_loop(0, _NCHUNK, chunk_body, 0)


@jax.jit
def _sc_decode(voxel, pixel, conf, off, view, vel):
    mesh = plsc.VectorSubcoreMesh(core_axis_name="c", subcore_axis_name="s")
    f = pl.kernel(
        _decode_body,
        out_type=jax.ShapeDtypeStruct((4, _B, _ROWS, _COLS), jnp.float32),
        mesh=mesh,
        compiler_params=pltpu.CompilerParams(
            needs_layout_passes=False, use_tc_tiling_on_sc=False),
        scratch_types=[
            pltpu.VMEM((_RC, _COLS), jnp.int32),
            pltpu.VMEM((_RC, _COLS), jnp.float32),
            pltpu.VMEM((_RC, _COLS), jnp.float32),
            pltpu.VMEM((_RC, _COLS), jnp.float32),
            pltpu.VMEM((_RC, _COLS), jnp.float32),
            pltpu.VMEM((_RC, _COLS), jnp.float32),
            pltpu.VMEM((_RC, _COLS), jnp.float32),
            pltpu.VMEM((_NV, _RC, _COLS), jnp.int32),
            pltpu.VMEM((4, _RC, _COLS), jnp.float32),
            pltpu.VMEM((_COLS,), jnp.float32),
        ],
    )
    planar = f(voxel, pixel, conf, off,
               view[..., 0], view[..., 1], view[..., 2], view[..., 3],
               view[..., 4], vel)
    return jnp.transpose(planar, (1, 2, 3, 0))


def kernel(voxel_count_gt, pixel_pred, confidence_pred, offset_pred,
           view_index, velocity_pred, fixedMem, fixedMem_float):
    return _sc_decode(voxel_count_gt, pixel_pred, confidence_pred,
                      offset_pred, view_index, velocity_pred)


# speedup vs baseline: 1.0354x; 1.0354x over previous
import jax
import jax.numpy as jnp
from jax import lax
from jax.experimental import pallas as pl
from jax.experimental.pallas import tpu as pltpu
from jax.experimental.pallas import tpu_sc as plsc

_B, _ROWS, _COLS = 4, 512, 512
_NV = 5
_EXT0, _EXT1 = -51.2, -51.2
_GRID_R = 102.4 / _ROWS
_GRID_C = 102.4 / _COLS
_THRESH = 0.05

_NC, _NS, _L = 2, 16, 16
_NW = _NC * _NS
_ROWS_PER_W = (_B * _ROWS) // _NW
_RC = 8
_NCHUNK = _ROWS_PER_W // _RC
_WPB = _ROWS // _ROWS_PER_W


def _rsqrt(x):
    i = lax.bitcast_convert_type(x, jnp.int32)
    i = jnp.int32(0x5F3759DF) - lax.shift_right_logical(i, 1)
    y = lax.bitcast_convert_type(i, jnp.float32)
    for _ in range(2):
        y = y * (1.5 - 0.5 * x * y * y)
    return y


def _decode_body(voxel, pixel, conf, off, view0, view1, view2, view3, view4,
                 vel, out,
                 voxel_b, pixel_b, conf_b, off0_b, off1_b, vel0_b, vel1_b,
                 view_b, out_b, ccol_b):
    views = (view0, view1, view2, view3, view4)
    wid = lax.axis_index("s") * _NC + lax.axis_index("c")
    b = wid // _WPB
    r_base = (wid % _WPB) * _ROWS_PER_W

    iota = lax.iota(jnp.int32, _L)
    iota_f = iota.astype(jnp.float32)

    def ccol_body(g, c):
        cb = g * _L
        ccol_b[pl.ds(cb, _L)] = _EXT1 + (cb + iota_f + 0.5) * _GRID_C
        return c
    lax.fori_loop(0, _COLS // _L, ccol_body, 0)

    def chunk_body(ci, carry):
        r0 = r_base + ci * _RC
        pltpu.sync_copy(voxel.at[b, pl.ds(r0, _RC)], voxel_b)
        pltpu.sync_copy(pixel.at[b, pl.ds(r0, _RC)], pixel_b)
        pltpu.sync_copy(conf.at[b, pl.ds(r0, _RC)], conf_b)
        pltpu.sync_copy(off.at[b, 0, pl.ds(r0, _RC)], off0_b)
        pltpu.sync_copy(off.at[b, 1, pl.ds(r0, _RC)], off1_b)
        pltpu.sync_copy(vel.at[b, 0, pl.ds(r0, _RC)], vel0_b)
        pltpu.sync_copy(vel.at[b, 1, pl.ds(r0, _RC)], vel1_b)
        for v in range(_NV):
            pltpu.sync_copy(views[v].at[b, pl.ds(r0, _RC)], view_b.at[v])

        for row in range(_RC):
            r_glob = (r0 + row).astype(jnp.float32)
            cr_base = _EXT0 + (r_glob + 0.5) * _GRID_R
            row_splat = jnp.full((_L,), row, jnp.int32)

            def group_body(g, carry2):
                cbase = g * _L
                c_vec = cbase + iota
                s = jnp.zeros((_L,), jnp.float32)
                for v in range(_NV):
                    idx_v = view_b[v, row, pl.ds(cbase, _L)]
                    s = s + plsc.load_gather(conf_b, [row_splat, idx_v])
                conf_lin = conf_b[row, pl.ds(cbase, _L)]
                conf_final = 0.5 * (conf_lin + s * (1.0 / _NV))

                center_r = cr_base + off0_b[row, pl.ds(cbase, _L)] * _GRID_R
                center_c = (ccol_b[pl.ds(cbase, _L)]
                            + off1_b[row, pl.ds(cbase, _L)] * _GRID_C)

                vx = vel0_b[row, pl.ds(cbase, _L)]
                vy = vel1_b[row, pl.ds(cbase, _L)]
                s2 = vx * vx + vy * vy + 1e-12
                speed = s2 * _rsqrt(s2)

                mask = ((pixel_b[row, pl.ds(cbase, _L)] > _THRESH)
                        & (voxel_b[row, pl.ds(cbase, _L)] > 0))
                neg = jnp.full((_L,), -0.1, jnp.float32)
                for ch, val in enumerate((center_r, center_c, conf_final,
                                          speed)):
                    out_b[ch, row, pl.ds(cbase, _L)] = jnp.where(mask, val,
                                                                 neg)
                return carry2

            lax.fori_loop(0, _COLS // _L, group_body, 0, unroll=2)

        for ch in range(4):
            pltpu.sync_copy(out_b.at[ch], out.at[ch, b, pl.ds(r0, _RC)])
        return carry

    lax.fori_loop(0, _NCHUNK, chunk_body, 0)


@jax.jit
def _sc_decode(voxel, pixel, conf, off, view, vel):
    mesh = plsc.VectorSubcoreMesh(core_axis_name="c", subcore_axis_name="s")
    f = pl.kernel(
        _decode_body,
        out_type=jax.ShapeDtypeStruct((4, _B, _ROWS, _COLS), jnp.float32),
        mesh=mesh,
        compiler_params=pltpu.CompilerParams(
            needs_layout_passes=False, use_tc_tiling_on_sc=False),
        scratch_types=[
            pltpu.VMEM((_RC, _COLS), jnp.int32),
            pltpu.VMEM((_RC, _COLS), jnp.float32),
            pltpu.VMEM((_RC, _COLS), jnp.float32),
            pltpu.VMEM((_RC, _COLS), jnp.float32),
            pltpu.VMEM((_RC, _COLS), jnp.float32),
            pltpu.VMEM((_RC, _COLS), jnp.float32),
            pltpu.VMEM((_RC, _COLS), jnp.float32),
            pltpu.VMEM((_NV, _RC, _COLS), jnp.int32),
            pltpu.VMEM((4, _RC, _COLS), jnp.float32),
            pltpu.VMEM((_COLS,), jnp.float32),
        ],
    )
    planar = f(voxel, pixel, conf, off,
               view[..., 0], view[..., 1], view[..., 2], view[..., 3],
               view[..., 4], vel)
    return jnp.transpose(planar, (1, 2, 3, 0))


def kernel(voxel_count_gt, pixel_pred, confidence_pred, offset_pred,
           view_index, velocity_pred, fixedMem, fixedMem_float):
    return _sc_decode(voxel_count_gt, pixel_pred, confidence_pred,
                      offset_pred, view_index, velocity_pred)


# speedup vs baseline: 1.6351x; 1.5792x over previous
import jax
import jax.numpy as jnp
from jax import lax
from jax.experimental import pallas as pl
from jax.experimental.pallas import tpu as pltpu
from jax.experimental.pallas import tpu_sc as plsc

_B, _ROWS, _COLS = 4, 512, 512
_NV = 5
_EXT0, _EXT1 = -51.2, -51.2
_GRID_R = 102.4 / _ROWS
_GRID_C = 102.4 / _COLS
_THRESH = 0.05

_NC, _NS, _L = 2, 16, 16
_NW = _NC * _NS
_ROWS_PER_W = (_B * _ROWS) // _NW
_RC = 4
_NCHUNK = _ROWS_PER_W // _RC
_WPB = _ROWS // _ROWS_PER_W


def _rsqrt(x):
    i = lax.bitcast_convert_type(x, jnp.int32)
    i = jnp.int32(0x5F3759DF) - lax.shift_right_logical(i, 1)
    y = lax.bitcast_convert_type(i, jnp.float32)
    for _ in range(2):
        y = y * (1.5 - 0.5 * x * y * y)
    return y


def _decode_body(voxel, pixel, conf, off, view0, view1, view2, view3, view4,
                 vel, out,
                 voxel_b, pixel_b, conf_b, off0_b, off1_b, vel0_b, vel1_b,
                 view_b, out_b, ccol_b, sem_in, sem_out):
    views = (view0, view1, view2, view3, view4)
    wid = lax.axis_index("s") * _NC + lax.axis_index("c")
    b = wid // _WPB
    r_base = (wid % _WPB) * _ROWS_PER_W

    iota = lax.iota(jnp.int32, _L)
    iota_f = iota.astype(jnp.float32)

    def ccol_body(g, c):
        cb = g * _L
        ccol_b[pl.ds(cb, _L)] = _EXT1 + (cb + iota_f + 0.5) * _GRID_C
        return c
    lax.fori_loop(0, _COLS // _L, ccol_body, 0)

    def in_pairs(ci, slot):
        r0 = r_base + ci * _RC
        p = [
            (voxel.at[b, pl.ds(r0, _RC)], voxel_b.at[slot]),
            (pixel.at[b, pl.ds(r0, _RC)], pixel_b.at[slot]),
            (conf.at[b, pl.ds(r0, _RC)], conf_b.at[slot]),
            (off.at[b, 0, pl.ds(r0, _RC)], off0_b.at[slot]),
            (off.at[b, 1, pl.ds(r0, _RC)], off1_b.at[slot]),
            (vel.at[b, 0, pl.ds(r0, _RC)], vel0_b.at[slot]),
            (vel.at[b, 1, pl.ds(r0, _RC)], vel1_b.at[slot]),
        ]
        for v in range(_NV):
            p.append((views[v].at[b, pl.ds(r0, _RC)], view_b.at[slot, v]))
        return p

    def issue_in(ci, slot, sem):
        for src, dst in in_pairs(ci, slot):
            pltpu.async_copy(src, dst, sem)

    def wait_in(ci, slot, sem):
        for src, dst in in_pairs(ci, slot):
            pltpu.make_async_copy(src, dst, sem).wait()

    def out_pairs(ci, slot):
        r0 = r_base + ci * _RC
        return [(out_b.at[slot, ch], out.at[ch, b, pl.ds(r0, _RC)])
                for ch in range(4)]

    def compute(ci, slot):
        r0 = r_base + ci * _RC
        for row in range(_RC):
            r_glob = (r0 + row).astype(jnp.float32)
            cr_base = _EXT0 + (r_glob + 0.5) * _GRID_R
            row_splat = jnp.full((_L,), row, jnp.int32)

            def group_body(g, carry2):
                cbase = g * _L
                s = jnp.zeros((_L,), jnp.float32)
                for v in range(_NV):
                    idx_v = view_b[slot, v, row, pl.ds(cbase, _L)]
                    s = s + plsc.load_gather(conf_b.at[slot],
                                             [row_splat, idx_v])
                conf_lin = conf_b[slot, row, pl.ds(cbase, _L)]
                conf_final = 0.5 * conf_lin + (0.5 / _NV) * s

                center_r = (cr_base
                            + off0_b[slot, row, pl.ds(cbase, _L)] * _GRID_R)
                center_c = (ccol_b[pl.ds(cbase, _L)]
                            + off1_b[slot, row, pl.ds(cbase, _L)] * _GRID_C)

                vx = vel0_b[slot, row, pl.ds(cbase, _L)]
                vy = vel1_b[slot, row, pl.ds(cbase, _L)]
                s2 = vx * vx + vy * vy + 1e-12
                speed = s2 * _rsqrt(s2)

                mask = ((pixel_b[slot, row, pl.ds(cbase, _L)] > _THRESH)
                        & (voxel_b[slot, row, pl.ds(cbase, _L)] > 0))
                neg = jnp.full((_L,), -0.1, jnp.float32)
                for ch, val in enumerate((center_r, center_c, conf_final,
                                          speed)):
                    out_b[slot, ch, row, pl.ds(cbase, _L)] = jnp.where(
                        mask, val, neg)
                return carry2

            lax.fori_loop(0, _COLS // _L, group_body, 0, unroll=2)

    issue_in(0, 0, sem_in)

    def pair_body(k, carry):
        ci_a = 2 * k
        ci_b = 2 * k + 1
        issue_in(ci_b, 1, sem_in)
        wait_in(ci_a, 0, sem_in)

        @pl.when(k > 0)
        def _():
            for src, dst in out_pairs(ci_a - 2, 0):
                pltpu.make_async_copy(src, dst, sem_out).wait()

        compute(ci_a, 0)
        for src, dst in out_pairs(ci_a, 0):
            pltpu.async_copy(src, dst, sem_out)

        @pl.when(k < _NCHUNK // 2 - 1)
        def _():
            issue_in(ci_a + 2, 0, sem_in)

        wait_in(ci_b, 1, sem_in)

        @pl.when(k > 0)
        def _():
            for src, dst in out_pairs(ci_b - 2, 1):
                pltpu.make_async_copy(src, dst, sem_out).wait()

        compute(ci_b, 1)
        for src, dst in out_pairs(ci_b, 1):
            pltpu.async_copy(src, dst, sem_out)
        return carry

    lax.fori_loop(0, _NCHUNK // 2, pair_body, 0)

    for src, dst in out_pairs(_NCHUNK - 2, 0):
        pltpu.make_async_copy(src, dst, sem_out).wait()
    for src, dst in out_pairs(_NCHUNK - 1, 1):
        pltpu.make_async_copy(src, dst, sem_out).wait()


@jax.jit
def _sc_decode(voxel, pixel, conf, off, view, vel):
    mesh = plsc.VectorSubcoreMesh(core_axis_name="c", subcore_axis_name="s")
    f = pl.kernel(
        _decode_body,
        out_type=jax.ShapeDtypeStruct((4, _B, _ROWS, _COLS), jnp.float32),
        mesh=mesh,
        compiler_params=pltpu.CompilerParams(
            needs_layout_passes=False, use_tc_tiling_on_sc=False),
        scratch_types=[
            pltpu.VMEM((2, _RC, _COLS), jnp.int32),
            pltpu.VMEM((2, _RC, _COLS), jnp.float32),
            pltpu.VMEM((2, _RC, _COLS), jnp.float32),
            pltpu.VMEM((2, _RC, _COLS), jnp.float32),
            pltpu.VMEM((2, _RC, _COLS), jnp.float32),
            pltpu.VMEM((2, _RC, _COLS), jnp.float32),
            pltpu.VMEM((2, _RC, _COLS), jnp.float32),
            pltpu.VMEM((2, _NV, _RC, _COLS), jnp.int32),
            pltpu.VMEM((2, 4, _RC, _COLS), jnp.float32),
            pltpu.VMEM((_COLS,), jnp.float32),
            pltpu.SemaphoreType.DMA,
            pltpu.SemaphoreType.DMA,
        ],
    )
    planar = f(voxel, pixel, conf, off,
               view[..., 0], view[..., 1], view[..., 2], view[..., 3],
               view[..., 4], vel)
    return jnp.transpose(planar, (1, 2, 3, 0))


def kernel(voxel_count_gt, pixel_pred, confidence_pred, offset_pred,
           view_index, velocity_pred, fixedMem, fixedMem_float):
    return _sc_decode(voxel_count_gt, pixel_pred, confidence_pred,
                      offset_pred, view_index, velocity_pred)
